# Initial kernel scaffold; baseline (speedup 1.0000x reference)
import jax
import jax.numpy as jnp
from jax.experimental import pallas as pl


def kernel(word, feat, W_word, W_feat):
    raise NotImplementedError("write your pallas kernel here")


# speedup vs baseline: 3.6222x; 3.6222x over previous
import functools

import jax
import jax.numpy as jnp
from jax import lax
from jax.experimental import pallas as pl
from jax.experimental.pallas import tpu as pltpu
from jax.experimental.pallas import tpu_sc as plsc

VOCAB_WORD = 100000
DIM_WORD = 128
VOCAB_FEAT = 1000
DIM_FEAT = 64
DIM_OUT = DIM_WORD + DIM_FEAT

NC = 2
NS = 16
NW = NC * NS

IDXW = 128
CHUNK = 256
GPC = CHUNK // IDXW


def _build(n_rows):
    assert n_rows % (NW * CHUNK) == 0
    rows_per_w = n_rows // NW
    n_chunks = rows_per_w // CHUNK

    mesh = plsc.VectorSubcoreMesh(core_axis_name="c", subcore_axis_name="s")

    @functools.partial(
        pl.kernel,
        mesh=mesh,
        out_type=jax.ShapeDtypeStruct((n_rows, DIM_OUT), jnp.float32),
        scratch_types=[
            pltpu.VMEM((rows_per_w,), jnp.int32),
            pltpu.VMEM((rows_per_w,), jnp.int32),
            pltpu.VMEM((CHUNK, DIM_WORD), jnp.float32),
            pltpu.VMEM((CHUNK, DIM_FEAT), jnp.float32),
            pltpu.SemaphoreType.DMA,
        ],
        compiler_params=pltpu.CompilerParams(use_tc_tiling_on_sc=False),
    )
    def k(word_hbm, feat_hbm, ww_hbm, wf_hbm, out_hbm,
          widx_v, fidx_v, bufw_v, buff_v, sem):
        wid = lax.axis_index("s") * NC + lax.axis_index("c")
        row0 = wid * rows_per_w
        pltpu.sync_copy(word_hbm.at[pl.ds(row0, rows_per_w)], widx_v)
        pltpu.sync_copy(feat_hbm.at[pl.ds(row0, rows_per_w)], fidx_v)

        def body(i, _):
            base = row0 + i * CHUNK
            for j in range(GPC):
                r = (i * GPC + j) * IDXW
                pltpu.async_copy(
                    ww_hbm.at[widx_v.at[pl.ds(r, IDXW)]],
                    bufw_v.at[pl.ds(j * IDXW, IDXW)], sem).wait()
                pltpu.async_copy(
                    wf_hbm.at[fidx_v.at[pl.ds(r, IDXW)]],
                    buff_v.at[pl.ds(j * IDXW, IDXW)], sem).wait()
            pltpu.sync_copy(bufw_v, out_hbm.at[pl.ds(base, CHUNK),
                                               pl.ds(0, DIM_WORD)])
            pltpu.sync_copy(buff_v, out_hbm.at[pl.ds(base, CHUNK),
                                               pl.ds(DIM_WORD, DIM_FEAT)])
            return _

        lax.fori_loop(0, n_chunks, body, 0)

    return k


def kernel(word, feat, W_word, W_feat):
    b, s = word.shape
    n_rows = b * s
    word1d = word.reshape(n_rows).astype(jnp.int32)
    feat1d = feat.reshape(n_rows).astype(jnp.int32)
    out = _build(n_rows)(word1d, feat1d, W_word, W_feat)
    return out.reshape(b, s, DIM_OUT)


# speedup vs baseline: 3.9772x; 1.0980x over previous
import functools

import jax
import jax.numpy as jnp
from jax import lax
from jax.experimental import pallas as pl
from jax.experimental.pallas import tpu as pltpu
from jax.experimental.pallas import tpu_sc as plsc

VOCAB_WORD = 100000
DIM_WORD = 128
VOCAB_FEAT = 1000
DIM_FEAT = 64
DIM_OUT = DIM_WORD + DIM_FEAT

NC = 2
NS = 16
NW = NC * NS

CHUNK = 128


def _build(n_rows):
    assert n_rows % (NW * CHUNK) == 0
    rows_per_w = n_rows // NW
    n_chunks = rows_per_w // CHUNK
    assert n_chunks % 2 == 0 or n_chunks >= 4

    mesh = plsc.VectorSubcoreMesh(core_axis_name="c", subcore_axis_name="s")

    @functools.partial(
        pl.kernel,
        mesh=mesh,
        out_type=jax.ShapeDtypeStruct((n_rows, DIM_OUT), jnp.float32),
        scratch_types=[
            pltpu.VMEM((rows_per_w,), jnp.int32),
            pltpu.VMEM((rows_per_w,), jnp.int32),
            pltpu.VMEM((2, CHUNK, DIM_WORD), jnp.float32),
            pltpu.VMEM((2, CHUNK, DIM_FEAT), jnp.float32),
            pltpu.SemaphoreType.DMA,
            pltpu.SemaphoreType.DMA,
            pltpu.SemaphoreType.DMA,
            pltpu.SemaphoreType.DMA,
        ],
        compiler_params=pltpu.CompilerParams(use_tc_tiling_on_sc=False),
    )
    def k(word_hbm, feat_hbm, ww_hbm, wf_hbm, out_hbm,
          widx_v, fidx_v, bufw_v, buff_v, gsem0, gsem1, wsem0, wsem1):
        wid = lax.axis_index("s") * NC + lax.axis_index("c")
        row0 = wid * rows_per_w
        pltpu.sync_copy(word_hbm.at[pl.ds(row0, rows_per_w)], widx_v)
        pltpu.sync_copy(feat_hbm.at[pl.ds(row0, rows_per_w)], fidx_v)

        gsem = (gsem0, gsem1)
        wsem = (wsem0, wsem1)

        def fire_gather(c, s):
            off = c * CHUNK
            pltpu.async_copy(ww_hbm.at[widx_v.at[pl.ds(off, CHUNK)]],
                             bufw_v.at[s], gsem[s])
            pltpu.async_copy(wf_hbm.at[fidx_v.at[pl.ds(off, CHUNK)]],
                             buff_v.at[s], gsem[s])

        def drain_gather(s):
            pltpu.make_async_copy(ww_hbm.at[pl.ds(0, CHUNK)],
                                  bufw_v.at[s], gsem[s]).wait()
            pltpu.make_async_copy(wf_hbm.at[pl.ds(0, CHUNK)],
                                  buff_v.at[s], gsem[s]).wait()

        def fire_write(c, s):
            base = row0 + c * CHUNK
            pltpu.async_copy(bufw_v.at[s],
                             out_hbm.at[pl.ds(base, CHUNK), pl.ds(0, DIM_WORD)],
                             wsem[s])
            pltpu.async_copy(buff_v.at[s],
                             out_hbm.at[pl.ds(base, CHUNK),
                                        pl.ds(DIM_WORD, DIM_FEAT)],
                             wsem[s])

        def drain_write(s):
            pltpu.make_async_copy(bufw_v.at[s],
                                  out_hbm.at[pl.ds(row0, CHUNK),
                                             pl.ds(0, DIM_WORD)],
                                  wsem[s]).wait()
            pltpu.make_async_copy(buff_v.at[s],
                                  out_hbm.at[pl.ds(row0, CHUNK),
                                             pl.ds(DIM_WORD, DIM_FEAT)],
                                  wsem[s]).wait()

        fire_gather(0, 0)
        drain_gather(0); fire_write(0, 0); fire_gather(1, 1)
        drain_gather(1); fire_write(1, 1); drain_write(0); fire_gather(2, 0)

        def body(k_, _):
            c0 = 2 * k_
            drain_gather(0); fire_write(c0, 0)
            drain_write(1); fire_gather(c0 + 1, 1)
            drain_gather(1); fire_write(c0 + 1, 1)
            drain_write(0); fire_gather(c0 + 2, 0)
            return _

        lax.fori_loop(1, n_chunks // 2 - 1, body, 0)

        c0 = n_chunks - 2
        drain_gather(0); fire_write(c0, 0)
        drain_write(1); fire_gather(c0 + 1, 1)
        drain_gather(1); fire_write(c0 + 1, 1)
        drain_write(0); drain_write(1)

    return k


def kernel(word, feat, W_word, W_feat):
    b, s = word.shape
    n_rows = b * s
    word1d = word.reshape(n_rows).astype(jnp.int32)
    feat1d = feat.reshape(n_rows).astype(jnp.int32)
    out = _build(n_rows)(word1d, feat1d, W_word, W_feat)
    return out.reshape(b, s, DIM_OUT)
